# baseline (speedup 1.0000x reference)
import functools

import jax
import jax.numpy as jnp
from jax import lax
from jax.experimental import pallas as pl
from jax.experimental.pallas import tpu as pltpu
import jax.experimental.pallas.tpu_sc as plsc

NC = 2
NS = 16
CH = 128


def _mm_body(x_ref, w_ref, o_ref):
    r = jnp.dot(x_ref[...], w_ref[...], preferred_element_type=jnp.float32)
    dh = r.shape[1] // 2
    o_ref[0] = r[:, :dh]
    o_ref[1] = r[:, dh:]


def _mid_body(agg_ref, y_ref, deg_ref, w_ref, b_ref, o_ref):
    a = jnp.concatenate([agg_ref[0] + y_ref[0], agg_ref[1] + y_ref[1]],
                        axis=1)
    deg = deg_ref[0][:, 0:1] + deg_ref[1][:, 0:1]
    h = jnp.maximum(a * (1.0 / (deg + 1.0)) + b_ref[...], 0.0)
    r = jnp.dot(h, w_ref[...], preferred_element_type=jnp.float32)
    dh = r.shape[1] // 2
    o_ref[0] = r[:, :dh]
    o_ref[1] = r[:, dh:]


def _fin_body(agg_ref, y_ref, deg_ref, b_ref, o_ref):
    a = jnp.concatenate([agg_ref[0] + y_ref[0], agg_ref[1] + y_ref[1]],
                        axis=1)
    deg = deg_ref[0][:, 0:1] + deg_ref[1][:, 0:1]
    o_ref[...] = a * (1.0 / (deg + 1.0)) + b_ref[...]


def _sc_agg_body(nch, rows, nbch, compute_deg, *refs):
    if compute_deg:
        (srcp, dstp, y, zc, ones_in, zrows,
         aggp, degp,
         sidx, didx, gb0, gb1, ones_b, dbuf,
         agg_sh, deg_sh, sg0, sg1, ss0, ss1, sd) = refs
    else:
        (srcp, dstp, y, zc,
         aggp,
         sidx, didx, gb0, gb1,
         agg_sh, sg0, sg1, ss0, ss1) = refs

    c = lax.axis_index("c")
    s = lax.axis_index("s")
    r0 = s * rows
    npairs = nch // 2
    deg_pairs = (npairs + 1) // 2

    pltpu.sync_copy(zc, gb0)
    for kk in range(nbch):
        pltpu.sync_copy(gb0, agg_sh.at[pl.ds(r0 + kk * CH, CH)])
    if compute_deg:
        pltpu.sync_copy(zrows, dbuf)
        pltpu.sync_copy(dbuf, deg_sh.at[pl.ds(r0, rows)])
        pltpu.sync_copy(ones_in, ones_b)

    pltpu.sync_copy(srcp.at[c, pl.ds(s * nch, nch)], sidx)
    pltpu.sync_copy(dstp.at[pl.ds(s * nch, nch)], didx)
    plsc.subcore_barrier()

    pltpu.async_copy(y.at[sidx.at[0]], gb0, sg0)
    pltpu.async_copy(y.at[sidx.at[1]], gb1, sg1)

    def pair(j, carry):
        b = 2 * j
        pltpu.make_async_copy(y.at[sidx.at[b]], gb0, sg0).wait()
        pltpu.async_copy(gb0, agg_sh.at[didx.at[b]], ss0, add=True)
        if compute_deg:
            do_deg = jnp.logical_or(jnp.logical_and(c == 0, j < deg_pairs),
                                    jnp.logical_and(c == 1, j >= deg_pairs))

            @pl.when(do_deg)
            def _():
                pltpu.async_copy(ones_b, deg_sh.at[didx.at[b]], sd, add=True)
                pltpu.async_copy(ones_b, deg_sh.at[didx.at[b + 1]], sd,
                                 add=True)

        pltpu.make_async_copy(y.at[sidx.at[b + 1]], gb1, sg1).wait()
        pltpu.async_copy(gb1, agg_sh.at[didx.at[b + 1]], ss1, add=True)

        @pl.when(j + 1 < npairs)
        def _():
            pltpu.make_async_copy(gb0, agg_sh.at[didx.at[b]], ss0).wait()
            pltpu.async_copy(y.at[sidx.at[b + 2]], gb0, sg0)
            pltpu.make_async_copy(gb1, agg_sh.at[didx.at[b + 1]], ss1).wait()
            pltpu.async_copy(y.at[sidx.at[b + 3]], gb1, sg1)

        return carry

    lax.fori_loop(0, npairs, pair, 0)
    pltpu.make_async_copy(gb0, agg_sh.at[didx.at[0]], ss0).wait()
    pltpu.make_async_copy(gb1, agg_sh.at[didx.at[1]], ss1).wait()
    if compute_deg:
        def drain(j, carry):
            pltpu.make_async_copy(ones_b, deg_sh.at[didx.at[0]], sd).wait()
            return carry
        lax.fori_loop(0, 2 * deg_pairs, drain, 0)
    plsc.subcore_barrier()

    for kk in range(nbch):
        pltpu.sync_copy(agg_sh.at[pl.ds(r0 + kk * CH, CH)], gb0)
        pltpu.sync_copy(gb0, aggp.at[c, pl.ds(r0 + kk * CH, CH)])
    if compute_deg:
        pltpu.sync_copy(deg_sh.at[pl.ds(r0, rows)], dbuf)
        pltpu.sync_copy(dbuf, degp.at[c, pl.ds(r0, rows)])


def kernel(inputs, edge_index, W1, b1, W2, b2):
    n, d = inputs.shape
    dh = d // 2
    e = edge_index.shape[1]

    rows = -(-n // (NS * CH)) * CH
    np_ = NS * rows
    nch = -(-e // (NS * CH))
    nch = (nch + 7) // 8 * 8
    e_pad = NS * nch * CH
    nbch = rows // CH

    npad = e_pad - e
    fill_src = jnp.arange(npad, dtype=jnp.int32) % 16
    fill_dst = n + fill_src
    src = jnp.concatenate([edge_index[0], fill_src]).reshape(NS * nch, CH)
    srcp = jnp.stack([src, src + n])
    dstp = jnp.concatenate([edge_index[1], fill_dst]).reshape(NS * nch, CH)
    zc = jnp.zeros((CH, dh), jnp.float32)
    ones_in = jnp.ones((CH, 16), jnp.float32)
    zrows = jnp.zeros((rows, 16), jnp.float32)
    b1r = b1.reshape(1, d)
    b2r = b2.reshape(1, d)

    BM = 1000
    grid = n // BM
    mm = pl.pallas_call(
        _mm_body, grid=(grid,),
        in_specs=[pl.BlockSpec((BM, d), lambda i: (i, 0)),
                  pl.BlockSpec((d, d), lambda i: (0, 0))],
        out_specs=pl.BlockSpec((NC, BM, dh), lambda i: (0, i, 0)),
        out_shape=jax.ShapeDtypeStruct((NC, n, dh), jnp.float32))

    mid = pl.pallas_call(
        _mid_body, grid=(grid,),
        in_specs=[pl.BlockSpec((NC, BM, dh), lambda i: (0, i, 0)),
                  pl.BlockSpec((NC, BM, dh), lambda i: (0, i, 0)),
                  pl.BlockSpec((NC, BM, 16), lambda i: (0, i, 0)),
                  pl.BlockSpec((d, d), lambda i: (0, 0)),
                  pl.BlockSpec((1, d), lambda i: (0, 0))],
        out_specs=pl.BlockSpec((NC, BM, dh), lambda i: (0, i, 0)),
        out_shape=jax.ShapeDtypeStruct((NC, n, dh), jnp.float32))

    fin = pl.pallas_call(
        _fin_body, grid=(grid,),
        in_specs=[pl.BlockSpec((NC, BM, dh), lambda i: (0, i, 0)),
                  pl.BlockSpec((NC, BM, dh), lambda i: (0, i, 0)),
                  pl.BlockSpec((NC, BM, 16), lambda i: (0, i, 0)),
                  pl.BlockSpec((1, d), lambda i: (0, 0))],
        out_specs=pl.BlockSpec((BM, d), lambda i: (i, 0)),
        out_shape=jax.ShapeDtypeStruct((n, d), jnp.float32))

    mesh = plsc.VectorSubcoreMesh(core_axis_name="c", subcore_axis_name="s",
                                  num_cores=NC, num_subcores=NS)

    def common_scratch():
        return [
            pltpu.VMEM((nch, CH), jnp.int32),
            pltpu.VMEM((nch, CH), jnp.int32),
            pltpu.VMEM((CH, dh), jnp.float32),
            pltpu.VMEM((CH, dh), jnp.float32),
        ]

    sc_params = pltpu.CompilerParams(use_tc_tiling_on_sc=False)
    agg_deg = pl.kernel(
        functools.partial(_sc_agg_body, nch, rows, nbch, True),
        out_type=(jax.ShapeDtypeStruct((NC, np_, dh), jnp.float32),
                  jax.ShapeDtypeStruct((NC, np_, 16), jnp.float32)),
        mesh=mesh,
        compiler_params=sc_params,
        scratch_types=common_scratch() + [
            pltpu.VMEM((CH, 16), jnp.float32),
            pltpu.VMEM((rows, 16), jnp.float32),
            pltpu.VMEM_SHARED((np_, dh), jnp.float32),
            pltpu.VMEM_SHARED((np_, 16), jnp.float32),
            pltpu.SemaphoreType.DMA,
            pltpu.SemaphoreType.DMA,
            pltpu.SemaphoreType.DMA,
            pltpu.SemaphoreType.DMA,
            pltpu.SemaphoreType.DMA,
        ])

    agg_only = pl.kernel(
        functools.partial(_sc_agg_body, nch, rows, nbch, False),
        out_type=jax.ShapeDtypeStruct((NC, np_, dh), jnp.float32),
        mesh=mesh,
        compiler_params=sc_params,
        scratch_types=common_scratch() + [
            pltpu.VMEM_SHARED((np_, dh), jnp.float32),
            pltpu.SemaphoreType.DMA,
            pltpu.SemaphoreType.DMA,
            pltpu.SemaphoreType.DMA,
            pltpu.SemaphoreType.DMA,
        ])

    y1 = mm(inputs, W1)
    y1f = y1.reshape(NC * n, dh)
    aggp1, degp = agg_deg(srcp, dstp, y1f, zc, ones_in, zrows)
    y2 = mid(aggp1, y1, degp, W2, b1r)
    y2f = y2.reshape(NC * n, dh)
    aggp2 = agg_only(srcp, dstp, y2f, zc)
    return fin(aggp2, y2, degp, b2r)


# speedup vs baseline: 1.1189x; 1.1189x over previous
import functools

import jax
import jax.numpy as jnp
from jax import lax
from jax.experimental import pallas as pl
from jax.experimental.pallas import tpu as pltpu
import jax.experimental.pallas.tpu_sc as plsc

NC = 2
NS = 16
CH = 128


def _mm_body(x_ref, w_ref, o_ref):
    r = jnp.dot(x_ref[...], w_ref[...], preferred_element_type=jnp.float32)
    dh = r.shape[1] // 2
    o_ref[0] = r[:, :dh]
    o_ref[1] = r[:, dh:]


def _mid_body(agg_ref, y_ref, deg_ref, w_ref, b_ref, o_ref):
    a = jnp.concatenate([agg_ref[0] + y_ref[0], agg_ref[1] + y_ref[1]],
                        axis=1)
    deg = deg_ref[0][:, 0:1] + deg_ref[1][:, 0:1]
    h = jnp.maximum(a * (1.0 / (deg + 1.0)) + b_ref[...], 0.0)
    r = jnp.dot(h, w_ref[...], preferred_element_type=jnp.float32)
    dh = r.shape[1] // 2
    o_ref[0] = r[:, :dh]
    o_ref[1] = r[:, dh:]


def _fin_body(agg_ref, y_ref, deg_ref, b_ref, o_ref):
    a = jnp.concatenate([agg_ref[0] + y_ref[0], agg_ref[1] + y_ref[1]],
                        axis=1)
    deg = deg_ref[0][:, 0:1] + deg_ref[1][:, 0:1]
    o_ref[...] = a * (1.0 / (deg + 1.0)) + b_ref[...]


def _sc_agg_body(nch, rows, nbch, compute_deg, *refs):
    if compute_deg:
        (srcp, dstp, y, zc, ones_in, zrows,
         aggp, degp,
         sidx, didx, gb0, gb1, ones_b, dbuf,
         agg_sh, deg_sh, sg0, sg1) = refs
    else:
        (srcp, dstp, y, zc,
         aggp,
         sidx, didx, gb0, gb1,
         agg_sh, sg0, sg1) = refs

    c = lax.axis_index("c")
    s = lax.axis_index("s")
    r0 = s * rows
    npairs = nch // 2
    deg_pairs = (npairs + 1) // 2

    pltpu.sync_copy(zc, gb0)
    for kk in range(nbch):
        pltpu.sync_copy(gb0, agg_sh.at[pl.ds(r0 + kk * CH, CH)])
    if compute_deg:
        pltpu.sync_copy(zrows, dbuf)
        pltpu.sync_copy(dbuf, deg_sh.at[pl.ds(r0, rows)])
        pltpu.sync_copy(ones_in, ones_b)

    pltpu.sync_copy(srcp.at[c, pl.ds(s * nch, nch)], sidx)
    pltpu.sync_copy(dstp.at[pl.ds(s * nch, nch)], didx)
    plsc.subcore_barrier()

    pltpu.async_copy(y.at[sidx.at[0]], gb0, sg0)

    def pair(j, carry):
        b = 2 * j
        pltpu.async_copy(y.at[sidx.at[b + 1]], gb1, sg1)
        pltpu.make_async_copy(y.at[sidx.at[b]], gb0, sg0).wait()
        pltpu.sync_copy(gb0, agg_sh.at[didx.at[b]], add=True)
        if compute_deg:
            do_deg = jnp.logical_or(jnp.logical_and(c == 0, j < deg_pairs),
                                    jnp.logical_and(c == 1, j >= deg_pairs))

            @pl.when(do_deg)
            def _():
                pltpu.sync_copy(ones_b, deg_sh.at[didx.at[b]], add=True)
                pltpu.sync_copy(ones_b, deg_sh.at[didx.at[b + 1]], add=True)

        @pl.when(j + 1 < npairs)
        def _():
            pltpu.async_copy(y.at[sidx.at[b + 2]], gb0, sg0)

        pltpu.make_async_copy(y.at[sidx.at[b + 1]], gb1, sg1).wait()
        pltpu.sync_copy(gb1, agg_sh.at[didx.at[b + 1]], add=True)
        return carry

    lax.fori_loop(0, npairs, pair, 0)
    plsc.subcore_barrier()

    for kk in range(nbch):
        pltpu.sync_copy(agg_sh.at[pl.ds(r0 + kk * CH, CH)], gb0)
        pltpu.sync_copy(gb0, aggp.at[c, pl.ds(r0 + kk * CH, CH)])
    if compute_deg:
        pltpu.sync_copy(deg_sh.at[pl.ds(r0, rows)], dbuf)
        pltpu.sync_copy(dbuf, degp.at[c, pl.ds(r0, rows)])


def kernel(inputs, edge_index, W1, b1, W2, b2):
    n, d = inputs.shape
    dh = d // 2
    e = edge_index.shape[1]

    rows = -(-n // (NS * CH)) * CH
    np_ = NS * rows
    nch = -(-e // (NS * CH))
    nch = (nch + 7) // 8 * 8
    e_pad = NS * nch * CH
    nbch = rows // CH

    npad = e_pad - e
    fill_src = jnp.arange(npad, dtype=jnp.int32) % 16
    fill_dst = n + fill_src
    src = jnp.concatenate([edge_index[0], fill_src]).reshape(NS * nch, CH)
    srcp = jnp.stack([src, src + n])
    dstp = jnp.concatenate([edge_index[1], fill_dst]).reshape(NS * nch, CH)
    zc = jnp.zeros((CH, dh), jnp.float32)
    ones_in = jnp.ones((CH, 16), jnp.float32)
    zrows = jnp.zeros((rows, 16), jnp.float32)
    b1r = b1.reshape(1, d)
    b2r = b2.reshape(1, d)

    BM = 1000
    grid = n // BM
    mm = pl.pallas_call(
        _mm_body, grid=(grid,),
        in_specs=[pl.BlockSpec((BM, d), lambda i: (i, 0)),
                  pl.BlockSpec((d, d), lambda i: (0, 0))],
        out_specs=pl.BlockSpec((NC, BM, dh), lambda i: (0, i, 0)),
        out_shape=jax.ShapeDtypeStruct((NC, n, dh), jnp.float32))

    mid = pl.pallas_call(
        _mid_body, grid=(grid,),
        in_specs=[pl.BlockSpec((NC, BM, dh), lambda i: (0, i, 0)),
                  pl.BlockSpec((NC, BM, dh), lambda i: (0, i, 0)),
                  pl.BlockSpec((NC, BM, 16), lambda i: (0, i, 0)),
                  pl.BlockSpec((d, d), lambda i: (0, 0)),
                  pl.BlockSpec((1, d), lambda i: (0, 0))],
        out_specs=pl.BlockSpec((NC, BM, dh), lambda i: (0, i, 0)),
        out_shape=jax.ShapeDtypeStruct((NC, n, dh), jnp.float32))

    fin = pl.pallas_call(
        _fin_body, grid=(grid,),
        in_specs=[pl.BlockSpec((NC, BM, dh), lambda i: (0, i, 0)),
                  pl.BlockSpec((NC, BM, dh), lambda i: (0, i, 0)),
                  pl.BlockSpec((NC, BM, 16), lambda i: (0, i, 0)),
                  pl.BlockSpec((1, d), lambda i: (0, 0))],
        out_specs=pl.BlockSpec((BM, d), lambda i: (i, 0)),
        out_shape=jax.ShapeDtypeStruct((n, d), jnp.float32))

    mesh = plsc.VectorSubcoreMesh(core_axis_name="c", subcore_axis_name="s",
                                  num_cores=NC, num_subcores=NS)

    def common_scratch():
        return [
            pltpu.VMEM((nch, CH), jnp.int32),
            pltpu.VMEM((nch, CH), jnp.int32),
            pltpu.VMEM((CH, dh), jnp.float32),
            pltpu.VMEM((CH, dh), jnp.float32),
        ]

    sc_params = pltpu.CompilerParams(use_tc_tiling_on_sc=False)
    agg_deg = pl.kernel(
        functools.partial(_sc_agg_body, nch, rows, nbch, True),
        out_type=(jax.ShapeDtypeStruct((NC, np_, dh), jnp.float32),
                  jax.ShapeDtypeStruct((NC, np_, 16), jnp.float32)),
        mesh=mesh,
        compiler_params=sc_params,
        scratch_types=common_scratch() + [
            pltpu.VMEM((CH, 16), jnp.float32),
            pltpu.VMEM((rows, 16), jnp.float32),
            pltpu.VMEM_SHARED((np_, dh), jnp.float32),
            pltpu.VMEM_SHARED((np_, 16), jnp.float32),
            pltpu.SemaphoreType.DMA,
            pltpu.SemaphoreType.DMA,
        ])

    agg_only = pl.kernel(
        functools.partial(_sc_agg_body, nch, rows, nbch, False),
        out_type=jax.ShapeDtypeStruct((NC, np_, dh), jnp.float32),
        mesh=mesh,
        compiler_params=sc_params,
        scratch_types=common_scratch() + [
            pltpu.VMEM_SHARED((np_, dh), jnp.float32),
            pltpu.SemaphoreType.DMA,
            pltpu.SemaphoreType.DMA,
        ])

    y1 = mm(inputs, W1)
    y1f = y1.reshape(NC * n, dh)
    aggp1, degp = agg_deg(srcp, dstp, y1f, zc, ones_in, zrows)
    y2 = mid(aggp1, y1, degp, W2, b1r)
    y2f = y2.reshape(NC * n, dh)
    aggp2 = agg_only(srcp, dstp, y2f, zc)
    return fin(aggp2, y2, degp, b2r)


# speedup vs baseline: 1.1530x; 1.0305x over previous
import functools

import jax
import jax.numpy as jnp
from jax import lax
from jax.experimental import pallas as pl
from jax.experimental.pallas import tpu as pltpu
import jax.experimental.pallas.tpu_sc as plsc

NC = 2
NS = 16
CH = 128


def _mm_body(x_ref, w_ref, o_ref):
    r = jnp.dot(x_ref[...], w_ref[...], preferred_element_type=jnp.float32)
    dh = r.shape[1] // 2
    o_ref[0] = r[:, :dh]
    o_ref[1] = r[:, dh:]


def _mid_body(agg_ref, y_ref, deg_ref, w_ref, b_ref, o_ref):
    a = jnp.concatenate([agg_ref[0] + y_ref[0], agg_ref[1] + y_ref[1]],
                        axis=1)
    deg = deg_ref[0][:, 0:1] + deg_ref[1][:, 0:1]
    h = jnp.maximum(a * (1.0 / (deg + 1.0)) + b_ref[...], 0.0)
    r = jnp.dot(h, w_ref[...], preferred_element_type=jnp.float32)
    dh = r.shape[1] // 2
    o_ref[0] = r[:, :dh]
    o_ref[1] = r[:, dh:]


def _fin_body(agg_ref, y_ref, deg_ref, b_ref, o_ref):
    a = jnp.concatenate([agg_ref[0] + y_ref[0], agg_ref[1] + y_ref[1]],
                        axis=1)
    deg = deg_ref[0][:, 0:1] + deg_ref[1][:, 0:1]
    o_ref[...] = a * (1.0 / (deg + 1.0)) + b_ref[...]


def _sc_agg_body(nch, rows, nbch, srcp, dstp, y, zc, degp_dep, aggp,
                 sidx, didx, gb0, gb1, agg_sh, sg0, sg1):
    c = lax.axis_index("c")
    s = lax.axis_index("s")
    r0 = s * rows
    npairs = nch // 2

    pltpu.sync_copy(zc, gb0)
    for kk in range(nbch):
        pltpu.sync_copy(gb0, agg_sh.at[pl.ds(r0 + kk * CH, CH)])

    pltpu.sync_copy(srcp.at[c, pl.ds(s * nch, nch)], sidx)
    pltpu.sync_copy(dstp.at[pl.ds(s * nch, nch)], didx)
    plsc.subcore_barrier()

    pltpu.async_copy(y.at[sidx.at[0]], gb0, sg0)

    def pair(j, carry):
        b = 2 * j
        pltpu.async_copy(y.at[sidx.at[b + 1]], gb1, sg1)
        pltpu.make_async_copy(y.at[sidx.at[b]], gb0, sg0).wait()
        pltpu.sync_copy(gb0, agg_sh.at[didx.at[b]], add=True)

        @pl.when(j + 1 < npairs)
        def _():
            pltpu.async_copy(y.at[sidx.at[b + 2]], gb0, sg0)

        pltpu.make_async_copy(y.at[sidx.at[b + 1]], gb1, sg1).wait()
        pltpu.sync_copy(gb1, agg_sh.at[didx.at[b + 1]], add=True)
        return carry

    lax.fori_loop(0, npairs, pair, 0)
    plsc.subcore_barrier()

    for kk in range(nbch):
        pltpu.sync_copy(agg_sh.at[pl.ds(r0 + kk * CH, CH)], gb0)
        pltpu.sync_copy(gb0, aggp.at[c, pl.ds(r0 + kk * CH, CH)])


def _sc_deg_body(nch, rows, dstp, ones_in, zrows, degp,
                 didx, ones_b, dbuf, deg_sh):
    c = lax.axis_index("c")
    s = lax.axis_index("s")
    r0 = s * rows
    half = nch // 2

    pltpu.sync_copy(zrows, dbuf)
    pltpu.sync_copy(dbuf, deg_sh.at[pl.ds(r0, rows)])
    pltpu.sync_copy(ones_in, ones_b)
    pltpu.sync_copy(dstp.at[pl.ds(s * nch, nch)], didx)
    plsc.subcore_barrier()

    def chunk(j, carry):
        pltpu.sync_copy(ones_b, deg_sh.at[didx.at[c * half + j]], add=True)
        return carry

    lax.fori_loop(0, half, chunk, 0)
    plsc.subcore_barrier()

    pltpu.sync_copy(deg_sh.at[pl.ds(r0, rows)], dbuf)
    pltpu.sync_copy(dbuf, degp.at[c, pl.ds(r0, rows)])


def kernel(inputs, edge_index, W1, b1, W2, b2):
    n, d = inputs.shape
    dh = d // 2
    e = edge_index.shape[1]

    rows = -(-n // (NS * CH)) * CH
    np_ = NS * rows
    nch = -(-e // (NS * CH))
    nch = (nch + 7) // 8 * 8
    e_pad = NS * nch * CH
    nbch = rows // CH

    npad = e_pad - e
    fill_src = jnp.arange(npad, dtype=jnp.int32) % 16
    fill_dst = n + fill_src
    src = jnp.concatenate([edge_index[0], fill_src]).reshape(NS * nch, CH)
    srcp = jnp.stack([src, src + n])
    dstp = jnp.concatenate([edge_index[1], fill_dst]).reshape(NS * nch, CH)
    zc = jnp.zeros((CH, dh), jnp.float32)
    ones_in = jnp.ones((CH, 16), jnp.float32)
    zrows = jnp.zeros((rows, 16), jnp.float32)
    b1r = b1.reshape(1, d)
    b2r = b2.reshape(1, d)

    BM = 1000
    grid = n // BM
    mm = pl.pallas_call(
        _mm_body, grid=(grid,),
        in_specs=[pl.BlockSpec((BM, d), lambda i: (i, 0)),
                  pl.BlockSpec((d, d), lambda i: (0, 0))],
        out_specs=pl.BlockSpec((NC, BM, dh), lambda i: (0, i, 0)),
        out_shape=jax.ShapeDtypeStruct((NC, n, dh), jnp.float32))

    mid = pl.pallas_call(
        _mid_body, grid=(grid,),
        in_specs=[pl.BlockSpec((NC, BM, dh), lambda i: (0, i, 0)),
                  pl.BlockSpec((NC, BM, dh), lambda i: (0, i, 0)),
                  pl.BlockSpec((NC, BM, 16), lambda i: (0, i, 0)),
                  pl.BlockSpec((d, d), lambda i: (0, 0)),
                  pl.BlockSpec((1, d), lambda i: (0, 0))],
        out_specs=pl.BlockSpec((NC, BM, dh), lambda i: (0, i, 0)),
        out_shape=jax.ShapeDtypeStruct((NC, n, dh), jnp.float32))

    fin = pl.pallas_call(
        _fin_body, grid=(grid,),
        in_specs=[pl.BlockSpec((NC, BM, dh), lambda i: (0, i, 0)),
                  pl.BlockSpec((NC, BM, dh), lambda i: (0, i, 0)),
                  pl.BlockSpec((NC, BM, 16), lambda i: (0, i, 0)),
                  pl.BlockSpec((1, d), lambda i: (0, 0))],
        out_specs=pl.BlockSpec((BM, d), lambda i: (i, 0)),
        out_shape=jax.ShapeDtypeStruct((n, d), jnp.float32))

    mesh = plsc.VectorSubcoreMesh(core_axis_name="c", subcore_axis_name="s",
                                  num_cores=NC, num_subcores=NS)

    def common_scratch():
        return [
            pltpu.VMEM((nch, CH), jnp.int32),
            pltpu.VMEM((nch, CH), jnp.int32),
            pltpu.VMEM((CH, dh), jnp.float32),
            pltpu.VMEM((CH, dh), jnp.float32),
        ]

    sc_params = pltpu.CompilerParams(use_tc_tiling_on_sc=False)
    agg = pl.kernel(
        functools.partial(_sc_agg_body, nch, rows, nbch),
        out_type=jax.ShapeDtypeStruct((NC, np_, dh), jnp.float32),
        mesh=mesh,
        compiler_params=sc_params,
        scratch_types=common_scratch() + [
            pltpu.VMEM_SHARED((np_, dh), jnp.float32),
            pltpu.SemaphoreType.DMA,
            pltpu.SemaphoreType.DMA,
        ])

    deg_kernel = pl.kernel(
        functools.partial(_sc_deg_body, nch, rows),
        out_type=jax.ShapeDtypeStruct((NC, np_, 16), jnp.float32),
        mesh=mesh,
        compiler_params=sc_params,
        scratch_types=[
            pltpu.VMEM((nch, CH), jnp.int32),
            pltpu.VMEM((CH, 16), jnp.float32),
            pltpu.VMEM((rows, 16), jnp.float32),
            pltpu.VMEM_SHARED((np_, 16), jnp.float32),
        ])

    degp = deg_kernel(dstp, ones_in, zrows)
    y1 = mm(inputs, W1)
    y1f = y1.reshape(NC * n, dh)
    aggp1 = agg(srcp, dstp, y1f, zc, degp)
    y2 = mid(aggp1, y1, degp, W2, b1r)
    y2f = y2.reshape(NC * n, dh)
    aggp2 = agg(srcp, dstp, y2f, zc, degp)
    return fin(aggp2, y2, degp, b2r)


# speedup vs baseline: 1.1622x; 1.0080x over previous
import functools

import jax
import jax.numpy as jnp
from jax import lax
from jax.experimental import pallas as pl
from jax.experimental.pallas import tpu as pltpu
import jax.experimental.pallas.tpu_sc as plsc

NC = 2
NS = 16
CH = 128


def _mm_body(x_ref, w_ref, o_ref):
    r = jnp.dot(x_ref[...], w_ref[...], preferred_element_type=jnp.float32)
    dh = r.shape[1] // 2
    o_ref[0] = r[:, :dh]
    o_ref[1] = r[:, dh:]


def _mid_body(agg_ref, y_ref, deg_ref, w_ref, b_ref, o_ref):
    a = jnp.concatenate([agg_ref[0] + y_ref[0], agg_ref[1] + y_ref[1]],
                        axis=1)
    deg = deg_ref[0][:, 0:1] + deg_ref[1][:, 0:1]
    h = jnp.maximum(a * (1.0 / (deg + 1.0)) + b_ref[...], 0.0)
    r = jnp.dot(h, w_ref[...], preferred_element_type=jnp.float32)
    dh = r.shape[1] // 2
    o_ref[0] = r[:, :dh]
    o_ref[1] = r[:, dh:]


def _fin_body(agg_ref, y_ref, deg_ref, b_ref, o_ref):
    a = jnp.concatenate([agg_ref[0] + y_ref[0], agg_ref[1] + y_ref[1]],
                        axis=1)
    deg = deg_ref[0][:, 0:1] + deg_ref[1][:, 0:1]
    o_ref[...] = a * (1.0 / (deg + 1.0)) + b_ref[...]


def _sc_agg_body(nch, rows, nbch, srcp, dstp, y, zc, degp_dep, aggp,
                 sidx, didx, gb0, gb1, agg_sh, sg0, sg1):
    c = lax.axis_index("c")
    s = lax.axis_index("s")
    r0 = s * rows
    npairs = nch // 2

    pltpu.sync_copy(zc, gb0)
    for kk in range(nbch):
        pltpu.sync_copy(gb0, agg_sh.at[pl.ds(r0 + kk * CH, CH)])

    pltpu.sync_copy(srcp.at[pl.ds(s * nch, nch)], sidx)
    pltpu.sync_copy(dstp.at[pl.ds(s * nch, nch)], didx)
    plsc.subcore_barrier()

    nsrc = y.shape[0] // NC
    ysl = y.at[pl.ds(c * nsrc, nsrc)]

    pltpu.async_copy(ysl.at[sidx.at[0]], gb0, sg0)

    def pair(j, carry):
        b = 2 * j
        pltpu.async_copy(ysl.at[sidx.at[b + 1]], gb1, sg1)
        pltpu.make_async_copy(ysl.at[sidx.at[b]], gb0, sg0).wait()
        pltpu.sync_copy(gb0, agg_sh.at[didx.at[b]], add=True)

        @pl.when(j + 1 < npairs)
        def _():
            pltpu.async_copy(ysl.at[sidx.at[b + 2]], gb0, sg0)

        pltpu.make_async_copy(ysl.at[sidx.at[b + 1]], gb1, sg1).wait()
        pltpu.sync_copy(gb1, agg_sh.at[didx.at[b + 1]], add=True)
        return carry

    lax.fori_loop(0, npairs, pair, 0)
    plsc.subcore_barrier()

    for kk in range(nbch):
        pltpu.sync_copy(agg_sh.at[pl.ds(r0 + kk * CH, CH)], gb0)
        pltpu.sync_copy(gb0, aggp.at[c, pl.ds(r0 + kk * CH, CH)])


def _sc_deg_body(nch, rows, dstp, ones_in, zrows, degp,
                 didx, ones_b, dbuf, deg_sh):
    c = lax.axis_index("c")
    s = lax.axis_index("s")
    r0 = s * rows
    half = nch // 2

    pltpu.sync_copy(zrows, dbuf)
    pltpu.sync_copy(dbuf, deg_sh.at[pl.ds(r0, rows)])
    pltpu.sync_copy(ones_in, ones_b)
    pltpu.sync_copy(dstp.at[pl.ds(s * nch, nch)], didx)
    plsc.subcore_barrier()

    def chunk(j, carry):
        pltpu.sync_copy(ones_b, deg_sh.at[didx.at[c * half + j]], add=True)
        return carry

    lax.fori_loop(0, half, chunk, 0)
    plsc.subcore_barrier()

    pltpu.sync_copy(deg_sh.at[pl.ds(r0, rows)], dbuf)
    pltpu.sync_copy(dbuf, degp.at[c, pl.ds(r0, rows)])


def kernel(inputs, edge_index, W1, b1, W2, b2):
    n, d = inputs.shape
    dh = d // 2
    e = edge_index.shape[1]

    rows = -(-n // (NS * CH)) * CH
    np_ = NS * rows
    nch = -(-e // (NS * CH))
    nch = (nch + 7) // 8 * 8
    e_pad = NS * nch * CH
    nbch = rows // CH

    npad = e_pad - e
    fill_src = jnp.arange(npad, dtype=jnp.int32) % 16
    fill_dst = n + fill_src
    srcp = jnp.concatenate([edge_index[0], fill_src]).reshape(NS * nch, CH)
    dstp = jnp.concatenate([edge_index[1], fill_dst]).reshape(NS * nch, CH)
    zc = jnp.zeros((CH, dh), jnp.float32)
    ones_in = jnp.ones((CH, 16), jnp.float32)
    zrows = jnp.zeros((rows, 16), jnp.float32)
    b1r = b1.reshape(1, d)
    b2r = b2.reshape(1, d)

    BM = 1000
    grid = n // BM
    mm = pl.pallas_call(
        _mm_body, grid=(grid,),
        in_specs=[pl.BlockSpec((BM, d), lambda i: (i, 0)),
                  pl.BlockSpec((d, d), lambda i: (0, 0))],
        out_specs=pl.BlockSpec((NC, BM, dh), lambda i: (0, i, 0)),
        out_shape=jax.ShapeDtypeStruct((NC, n, dh), jnp.float32))

    mid = pl.pallas_call(
        _mid_body, grid=(grid,),
        in_specs=[pl.BlockSpec((NC, BM, dh), lambda i: (0, i, 0)),
                  pl.BlockSpec((NC, BM, dh), lambda i: (0, i, 0)),
                  pl.BlockSpec((NC, BM, 16), lambda i: (0, i, 0)),
                  pl.BlockSpec((d, d), lambda i: (0, 0)),
                  pl.BlockSpec((1, d), lambda i: (0, 0))],
        out_specs=pl.BlockSpec((NC, BM, dh), lambda i: (0, i, 0)),
        out_shape=jax.ShapeDtypeStruct((NC, n, dh), jnp.float32))

    fin = pl.pallas_call(
        _fin_body, grid=(grid,),
        in_specs=[pl.BlockSpec((NC, BM, dh), lambda i: (0, i, 0)),
                  pl.BlockSpec((NC, BM, dh), lambda i: (0, i, 0)),
                  pl.BlockSpec((NC, BM, 16), lambda i: (0, i, 0)),
                  pl.BlockSpec((1, d), lambda i: (0, 0))],
        out_specs=pl.BlockSpec((BM, d), lambda i: (i, 0)),
        out_shape=jax.ShapeDtypeStruct((n, d), jnp.float32))

    mesh = plsc.VectorSubcoreMesh(core_axis_name="c", subcore_axis_name="s",
                                  num_cores=NC, num_subcores=NS)

    def common_scratch():
        return [
            pltpu.VMEM((nch, CH), jnp.int32),
            pltpu.VMEM((nch, CH), jnp.int32),
            pltpu.VMEM((CH, dh), jnp.float32),
            pltpu.VMEM((CH, dh), jnp.float32),
        ]

    sc_params = pltpu.CompilerParams(use_tc_tiling_on_sc=False)
    agg = pl.kernel(
        functools.partial(_sc_agg_body, nch, rows, nbch),
        out_type=jax.ShapeDtypeStruct((NC, np_, dh), jnp.float32),
        mesh=mesh,
        compiler_params=sc_params,
        scratch_types=common_scratch() + [
            pltpu.VMEM_SHARED((np_, dh), jnp.float32),
            pltpu.SemaphoreType.DMA,
            pltpu.SemaphoreType.DMA,
        ])

    deg_kernel = pl.kernel(
        functools.partial(_sc_deg_body, nch, rows),
        out_type=jax.ShapeDtypeStruct((NC, np_, 16), jnp.float32),
        mesh=mesh,
        compiler_params=sc_params,
        scratch_types=[
            pltpu.VMEM((nch, CH), jnp.int32),
            pltpu.VMEM((CH, 16), jnp.float32),
            pltpu.VMEM((rows, 16), jnp.float32),
            pltpu.VMEM_SHARED((np_, 16), jnp.float32),
        ])

    degp = deg_kernel(dstp, ones_in, zrows)
    y1 = mm(inputs, W1)
    y1f = y1.reshape(NC * n, dh)
    aggp1 = agg(srcp, dstp, y1f, zc, degp)
    y2 = mid(aggp1, y1, degp, W2, b1r)
    y2f = y2.reshape(NC * n, dh)
    aggp2 = agg(srcp, dstp, y2f, zc, degp)
    return fin(aggp2, y2, degp, b2r)
